# baseline (speedup 1.0000x reference)
import functools

import jax
import jax.numpy as jnp
from jax import lax
from jax.experimental import pallas as pl
from jax.experimental.pallas import tpu as pltpu
from jax.experimental.pallas import tpu_sc as plsc

B = 128
N = 32768
NC = 2
NS = 16
NW = NC * NS
ROWS_PER_W = B // NW
L = 16
U = 8
NG = N // (L * U)
BIG = 2**31 - 1


def _v_body(x_hbm, v_hbm, xb, vconst, gmax, semx, semv):
    wid = lax.axis_index("s") * NC + lax.axis_index("c")
    r0 = wid * ROWS_PER_W
    ii = lax.iota(jnp.int32, L)

    def refill(j, _):
        vconst[pl.ds(j * L, L)] = jnp.full((L,), -5.0, jnp.float32)
        return 0

    lax.fori_loop(0, N // L, refill, 0, unroll=4)

    hx = {}

    def issue_x(r):
        hx[r] = pltpu.async_copy(x_hbm.at[r0 + r], xb.at[r % 2],
                                 semx.at[r % 2])

    issue_x(0)
    prev_base = jnp.int32(0)
    prev_fb = jnp.bool_(False)
    for r in range(ROWS_PER_W):
        row = r0 + r
        b = r % 2
        if r + 1 < ROWS_PER_W:
            issue_x(r + 1)
        hx[r].wait()
        xrow = xb.at[b]

        def step(g, mxc):
            off0 = g * (L * U)
            xs = [xrow[pl.ds(off0 + u * L, L)] for u in range(U)]
            m01 = jnp.maximum(xs[0], xs[1])
            m23 = jnp.maximum(xs[2], xs[3])
            m45 = jnp.maximum(xs[4], xs[5])
            m67 = jnp.maximum(xs[6], xs[7])
            mg = jnp.maximum(jnp.maximum(m01, m23), jnp.maximum(m45, m67))
            gmax[pl.ds(g * L, L)] = mg
            return jnp.maximum(mxc, mg)

        mx = lax.fori_loop(0, NG, step,
                           jnp.full((L,), -jnp.inf, jnp.float32), unroll=2)
        m = mx[0]
        for j in range(1, L):
            m = jnp.maximum(m, mx[j])
        any_spike = m >= 4.0
        mvec = lax.broadcast_in_dim(m, (L,), ())

        def gstep(g, fnd):
            gv = gmax[pl.ds(g * L, L)]
            hit = jnp.logical_and(gv == mvec, fnd == BIG)
            return jnp.where(hit, jnp.full((L,), 1, jnp.int32) * g, fnd)

        fg = lax.fori_loop(0, NG, gstep,
                           jnp.full((L,), BIG, jnp.int32), unroll=4)
        gw = fg[0]
        for j in range(1, L):
            gw = jnp.minimum(gw, fg[j])

        def sstep(t, fnd):
            xv = xrow[pl.ds(gw * (L * U) + t * L, L)]
            hit = jnp.logical_and(xv == mvec, fnd == BIG)
            return jnp.where(hit, ii + t * L, fnd)

        fnd = lax.fori_loop(0, U, sstep, jnp.full((L,), BIG, jnp.int32))
        win_in = fnd[0]
        for j in range(1, L):
            win_in = jnp.minimum(win_in, fnd[j])
        win = gw * (L * U) + win_in
        base = (win // L) * L
        off = win - base

        if r > 0:
            pltpu.make_async_copy(x_hbm.at[row], vconst, semv).wait()

            @pl.when(prev_fb)
            def _():
                lax.fori_loop(0, N // L, refill, 0, unroll=4)

            @pl.when(jnp.logical_not(prev_fb))
            def _(pb=prev_base):
                vconst[pl.ds(pb, L)] = jnp.full((L,), -5.0, jnp.float32)

        @pl.when(any_spike)
        def _():
            vconst[pl.ds(base, L)] = jnp.where(
                ii == off, jnp.float32(0.0), jnp.float32(-5.0))
            pltpu.async_copy(vconst, v_hbm.at[row], semv)

        @pl.when(jnp.logical_not(any_spike))
        def _():
            def vstep(t, _):
                vconst[pl.ds(t * L, L)] = xrow[pl.ds(t * L, L)] * 0.25
                return 0

            lax.fori_loop(0, N // L, vstep, 0, unroll=4)
            pltpu.async_copy(vconst, v_hbm.at[row], semv)

        prev_base = base
        prev_fb = jnp.logical_not(any_spike)

    pltpu.make_async_copy(x_hbm.at[r0], vconst, semv).wait()


@jax.jit
def _lif_hybrid(x):
    f32 = jnp.float32

    def zi_body(x_ref, z_ref, i_ref):
        xv = x_ref[...]
        i_ref[...] = xv * 0.5
        z_ref[...] = jnp.where(xv >= 4.0, jnp.float32(1.0),
                               jnp.float32(0.0))

    TR = 32
    z, i = pl.pallas_call(
        zi_body,
        grid=(B // TR,),
        in_specs=[pl.BlockSpec((TR, N), lambda g: (g, 0))],
        out_specs=[pl.BlockSpec((TR, N), lambda g: (g, 0)),
                   pl.BlockSpec((TR, N), lambda g: (g, 0))],
        out_shape=[jax.ShapeDtypeStruct((B, N), f32),
                   jax.ShapeDtypeStruct((B, N), f32)],
    )(x)

    v = functools.partial(
        pl.kernel,
        mesh=plsc.VectorSubcoreMesh(core_axis_name="c", subcore_axis_name="s"),
        out_type=jax.ShapeDtypeStruct((B, N), f32),
        scratch_types=[
            pltpu.VMEM((2, N), f32),
            pltpu.VMEM((N,), f32),
            pltpu.VMEM((NG * L,), f32),
            pltpu.SemaphoreType.DMA((2,)),
            pltpu.SemaphoreType.DMA,
        ],
    )(_v_body)(x)

    return z, v, i


def kernel(x):
    z, new_v, i_new = _lif_hybrid(x)
    return z, new_v, i_new


# speedup vs baseline: 1.0765x; 1.0765x over previous
import functools

import jax
import jax.numpy as jnp
from jax import lax
from jax.experimental import pallas as pl
from jax.experimental.pallas import tpu as pltpu
from jax.experimental.pallas import tpu_sc as plsc

B = 128
N = 32768
NC = 2
NS = 16
NW = NC * NS
ROWS_PER_W = B // NW
L = 16
U = 8
NSEG = 4
SEG = N // NSEG
VPS = SEG // L
BIG = 2**31 - 1


def _butterfly(vec, op):
    ii = lax.iota(jnp.int32, L)
    for sh in (8, 4, 2, 1):
        vec = op(vec, vec[(ii + sh) % L])
    return vec


def _v_body(x_hbm, v_hbm, xb, vconst, semx, semv):
    wid = lax.axis_index("s") * NC + lax.axis_index("c")
    r0 = wid * ROWS_PER_W
    ii = lax.iota(jnp.int32, L)

    def refill(j, _):
        vconst[pl.ds(j * L, L)] = jnp.full((L,), -5.0, jnp.float32)
        return 0

    lax.fori_loop(0, N // L, refill, 0, unroll=4)

    hx = {}

    def issue_x(r):
        hx[r] = pltpu.async_copy(x_hbm.at[r0 + r], xb.at[r % 2],
                                 semx.at[r % 2])

    issue_x(0)
    prev_base = jnp.int32(0)
    prev_fb = jnp.bool_(False)
    for r in range(ROWS_PER_W):
        row = r0 + r
        b = r % 2
        if r + 1 < ROWS_PER_W:
            issue_x(r + 1)
        hx[r].wait()
        xrow = xb.at[b]

        smx = []
        for s in range(NSEG):
            def step(t, mxc, _s=s):
                off0 = _s * SEG + t * (L * U)
                xs = [xrow[pl.ds(off0 + u * L, L)] for u in range(U)]
                m01 = jnp.maximum(xs[0], xs[1])
                m23 = jnp.maximum(xs[2], xs[3])
                m45 = jnp.maximum(xs[4], xs[5])
                m67 = jnp.maximum(xs[6], xs[7])
                mg = jnp.maximum(jnp.maximum(m01, m23),
                                 jnp.maximum(m45, m67))
                return jnp.maximum(mxc, mg)

            smx.append(lax.fori_loop(
                0, VPS // U, step,
                jnp.full((L,), -jnp.inf, jnp.float32), unroll=1))

        sms = [_butterfly(smx[s], jnp.maximum)[0] for s in range(NSEG)]
        m = sms[0]
        for s in range(1, NSEG):
            m = jnp.maximum(m, sms[s])
        mvec = lax.broadcast_in_dim(m, (L,), ())
        any_spike = m >= 4.0

        sw = jnp.int32(NSEG - 1)
        for s in range(NSEG - 2, -1, -1):
            sw = jnp.where(sms[s] == m, jnp.int32(s), sw)
        seg0 = sw * SEG

        def sstep(t, fnd):
            xv = xrow[pl.ds(seg0 + t * L, L)]
            hit = jnp.logical_and(xv == mvec, fnd == BIG)
            return jnp.where(hit, ii + t * L, fnd)

        fnd = lax.fori_loop(0, VPS, sstep,
                            jnp.full((L,), BIG, jnp.int32), unroll=4)
        win_in = _butterfly(fnd, jnp.minimum)[0]
        win = seg0 + win_in
        base = (win // L) * L
        off = win - base

        if r > 0:
            pltpu.make_async_copy(x_hbm.at[row], vconst, semv).wait()

            @pl.when(prev_fb)
            def _():
                lax.fori_loop(0, N // L, refill, 0, unroll=4)

            @pl.when(jnp.logical_not(prev_fb))
            def _(pb=prev_base):
                vconst[pl.ds(pb, L)] = jnp.full((L,), -5.0, jnp.float32)

        @pl.when(any_spike)
        def _():
            vconst[pl.ds(base, L)] = jnp.where(
                ii == off, jnp.float32(0.0), jnp.float32(-5.0))
            pltpu.async_copy(vconst, v_hbm.at[row], semv)

        @pl.when(jnp.logical_not(any_spike))
        def _():
            def vstep(t, _):
                vconst[pl.ds(t * L, L)] = xrow[pl.ds(t * L, L)] * 0.25
                return 0

            lax.fori_loop(0, N // L, vstep, 0, unroll=4)
            pltpu.async_copy(vconst, v_hbm.at[row], semv)

        prev_base = base
        prev_fb = jnp.logical_not(any_spike)

    pltpu.make_async_copy(x_hbm.at[r0], vconst, semv).wait()


@jax.jit
def _lif_hybrid(x):
    f32 = jnp.float32

    def zi_body(x_ref, z_ref, i_ref):
        xv = x_ref[...]
        i_ref[...] = xv * 0.5
        z_ref[...] = jnp.where(xv >= 4.0, jnp.float32(1.0),
                               jnp.float32(0.0))

    TR = 32
    z, i = pl.pallas_call(
        zi_body,
        grid=(B // TR,),
        in_specs=[pl.BlockSpec((TR, N), lambda g: (g, 0))],
        out_specs=[pl.BlockSpec((TR, N), lambda g: (g, 0)),
                   pl.BlockSpec((TR, N), lambda g: (g, 0))],
        out_shape=[jax.ShapeDtypeStruct((B, N), f32),
                   jax.ShapeDtypeStruct((B, N), f32)],
    )(x)

    v = functools.partial(
        pl.kernel,
        mesh=plsc.VectorSubcoreMesh(core_axis_name="c", subcore_axis_name="s"),
        out_type=jax.ShapeDtypeStruct((B, N), f32),
        scratch_types=[
            pltpu.VMEM((2, N), f32),
            pltpu.VMEM((N,), f32),
            pltpu.SemaphoreType.DMA((2,)),
            pltpu.SemaphoreType.DMA,
        ],
    )(_v_body)(x)

    return z, v, i


def kernel(x):
    z, new_v, i_new = _lif_hybrid(x)
    return z, new_v, i_new
